# Initial kernel scaffold; baseline (speedup 1.0000x reference)
import jax
import jax.numpy as jnp
from jax.experimental import pallas as pl


def kernel(pc, feature, sa1_w0, sa1_b0, sa1_w1, sa1_b1, sa1_w2, sa1_b2, sa2_w0, sa2_b0):
    raise NotImplementedError("write your pallas kernel here")


# speedup vs baseline: 10.8593x; 10.8593x over previous
import functools

import jax
import jax.numpy as jnp
from jax import lax
from jax.experimental import pallas as pl
from jax.experimental.pallas import tpu as pltpu
from jax.experimental.pallas import tpu_sc as plsc

B, N, K, D = 8, 2048, 8, 128
RT = 128
NT = N // RT
F32 = jnp.float32
_HI = lax.Precision.HIGHEST


def _knn_proj_kernel(pc_all_ref, pc_rows_ref, xyzt_rows_ref, f_rows_ref,
                     wx1_ref, wf1_ref, idx_ref, g1_ref, p_ref):
    b = pl.program_id(0)
    xyz_all = pc_all_ref[0]
    a = pc_rows_ref[0]
    at = xyzt_rows_ref[0]
    f = f_rows_ref[0]
    sq_all = jnp.sum(xyz_all * xyz_all, axis=0, keepdims=True)
    sq_r = jnp.sum(at * at, axis=1, keepdims=True)
    inner = lax.dot_general(a.astype(jnp.bfloat16),
                            xyz_all.astype(jnp.bfloat16),
                            (((0,), (0,)), ((), ())),
                            preferred_element_type=F32)
    dist = (sq_r - 2.0 * inner) + sq_all
    col = lax.broadcasted_iota(jnp.int32, (RT, N), 1)
    base = b * N
    picks = []
    for _ in range(K):
        mn = jnp.min(dist, axis=1, keepdims=True)
        am = jnp.min(jnp.where(dist == mn, col, N), axis=1, keepdims=True)
        picks.append(am + base)
        dist = jnp.where(col == am, jnp.inf, dist)
    idx_ref[0] = jnp.concatenate(picks, axis=1)
    p = lax.dot_general(a, wx1_ref[...], (((0,), (0,)), ((), ())),
                        preferred_element_type=F32, precision=_HI)
    g = p + lax.dot_general(f, wf1_ref[...], (((0,), (0,)), ((), ())),
                            preferred_element_type=F32, precision=_HI)
    p_ref[0] = p
    g1_ref[0] = g


def _sa1_kernel(hg_ref, p_ref, pc_ref, w1_ref, w2_ref, b0_ref, b1_ref,
                b2_ref, wx2_ref, wf2_ref, r_ref, s_ref):
    hg = hg_ref[0]
    p = p_ref[0]
    h0 = hg - p[:, None, :] + b0_ref[...].reshape(1, 1, D)
    mu0 = jnp.mean(h0, axis=(0, 1), keepdims=True)
    inv0 = lax.rsqrt(jnp.mean((h0 - mu0) ** 2, axis=(0, 1), keepdims=True)
                     + 1e-5)
    a0 = jnp.maximum((h0 - mu0) * inv0, 0.0).reshape(N * K, D)
    h1 = jnp.dot(a0, w1_ref[...], preferred_element_type=F32,
                 precision=_HI) + b1_ref[...]
    mu1 = jnp.mean(h1, axis=0, keepdims=True)
    inv1 = lax.rsqrt(jnp.mean((h1 - mu1) ** 2, axis=0, keepdims=True) + 1e-5)
    a1 = jnp.maximum((h1 - mu1) * inv1, 0.0)
    h2 = jnp.dot(a1, w2_ref[...], preferred_element_type=F32,
                 precision=_HI) + b2_ref[...]
    mu2 = jnp.mean(h2, axis=0, keepdims=True)
    inv2 = lax.rsqrt(jnp.mean((h2 - mu2) ** 2, axis=0, keepdims=True) + 1e-5)
    m2 = jnp.max(h2.reshape(N, K, D), axis=1)
    fl1 = jnp.maximum((m2 - mu2) * inv2, 0.0)
    xyz = pc_ref[0]
    s = lax.dot_general(xyz, wx2_ref[...], (((0,), (0,)), ((), ())),
                        preferred_element_type=F32, precision=_HI)
    r = s + jnp.dot(fl1, wf2_ref[...], preferred_element_type=F32,
                    precision=_HI)
    r_ref[0] = r
    s_ref[0] = s


def _sa2_kernel(rg_ref, s_ref, b20_ref, out_ref):
    rg = rg_ref[0]
    s = s_ref[0]
    h = rg - s[:, None, :] + b20_ref[...].reshape(1, 1, D)
    mu = jnp.mean(h, axis=(0, 1), keepdims=True)
    inv = lax.rsqrt(jnp.mean((h - mu) ** 2, axis=(0, 1), keepdims=True) + 1e-5)
    m = jnp.max(h, axis=1)
    out_ref[0] = jnp.tanh((m - mu[0]) * inv[0])


def _sc_gather(table, idx2d):
    info = plsc.get_sparse_core_info()
    nw = info.num_cores * info.num_subcores
    chunks = idx2d.shape[0]
    cpw = chunks // nw
    mesh = plsc.VectorSubcoreMesh(core_axis_name="c", subcore_axis_name="s")

    @functools.partial(
        pl.kernel, mesh=mesh,
        out_type=jax.ShapeDtypeStruct((chunks * 128, D), F32),
        scratch_types=[
            pltpu.VMEM((cpw, 128), jnp.int32),
            pltpu.VMEM((128, D), F32),
            pltpu.SemaphoreType.DMA,
        ],
    )
    def k(table_hbm, idx_hbm, out_hbm, idx_v, rows_v, sem):
        wid = lax.axis_index("s") * info.num_cores + lax.axis_index("c")
        pltpu.sync_copy(idx_hbm.at[pl.ds(wid * cpw, cpw)], idx_v)

        def body(c, _):
            pltpu.async_copy(table_hbm.at[idx_v.at[c]], rows_v, sem).wait()
            pltpu.sync_copy(rows_v,
                            out_hbm.at[pl.ds((wid * cpw + c) * 128, 128)])
            return 0

        lax.fori_loop(0, cpw, body, 0)

    return k(table, idx2d)


def kernel(pc, feature, sa1_w0, sa1_b0, sa1_w1, sa1_b1, sa1_w2, sa1_b2,
           sa2_w0, sa2_b0):
    wx1, wf1 = sa1_w0[:3], sa1_w0[3:]
    wx2, wf2 = sa2_w0[:3], sa2_w0[3:]
    b0 = sa1_b0.reshape(1, D)
    b1 = sa1_b1.reshape(1, D)
    b2 = sa1_b2.reshape(1, D)
    b20 = sa2_b0.reshape(1, D)

    xyzt = jnp.transpose(pc, (0, 2, 1))
    idx, g1, p = pl.pallas_call(
        _knn_proj_kernel,
        grid=(B, NT),
        in_specs=[
            pl.BlockSpec((1, 3, N), lambda b, t: (b, 0, 0)),
            pl.BlockSpec((1, 3, RT), lambda b, t: (b, 0, t)),
            pl.BlockSpec((1, RT, 3), lambda b, t: (b, t, 0)),
            pl.BlockSpec((1, 64, RT), lambda b, t: (b, 0, t)),
            pl.BlockSpec((3, D), lambda b, t: (0, 0)),
            pl.BlockSpec((64, D), lambda b, t: (0, 0)),
        ],
        out_specs=[
            pl.BlockSpec((1, RT, K), lambda b, t: (b, t, 0)),
            pl.BlockSpec((1, RT, D), lambda b, t: (b, t, 0)),
            pl.BlockSpec((1, RT, D), lambda b, t: (b, t, 0)),
        ],
        out_shape=[
            jax.ShapeDtypeStruct((B, N, K), jnp.int32),
            jax.ShapeDtypeStruct((B, N, D), F32),
            jax.ShapeDtypeStruct((B, N, D), F32),
        ],
    )(pc, pc, xyzt, feature, wx1, wf1)

    idx2d = idx.reshape(B * N * K // 128, 128)
    hg = _sc_gather(g1.reshape(B * N, D), idx2d)

    full = lambda shape: pl.BlockSpec(shape, lambda b: (0,) * len(shape))
    r, s = pl.pallas_call(
        _sa1_kernel,
        grid=(B,),
        in_specs=[
            pl.BlockSpec((1, N, K, D), lambda b: (b, 0, 0, 0)),
            pl.BlockSpec((1, N, D), lambda b: (b, 0, 0)),
            pl.BlockSpec((1, 3, N), lambda b: (b, 0, 0)),
            full((D, D)), full((D, D)), full((1, D)), full((1, D)),
            full((1, D)), full((3, D)), full((D, D)),
        ],
        out_specs=[
            pl.BlockSpec((1, N, D), lambda b: (b, 0, 0)),
            pl.BlockSpec((1, N, D), lambda b: (b, 0, 0)),
        ],
        out_shape=[
            jax.ShapeDtypeStruct((B, N, D), F32),
            jax.ShapeDtypeStruct((B, N, D), F32),
        ],
    )(hg.reshape(B, N, K, D), p, pc, sa1_w1, sa1_w2, b0, b1, b2, wx2, wf2)

    rg = _sc_gather(r.reshape(B * N, D), idx2d)

    out = pl.pallas_call(
        _sa2_kernel,
        grid=(B,),
        in_specs=[
            pl.BlockSpec((1, N, K, D), lambda b: (b, 0, 0, 0)),
            pl.BlockSpec((1, N, D), lambda b: (b, 0, 0)),
            full((1, D)),
        ],
        out_specs=pl.BlockSpec((1, N, D), lambda b: (b, 0, 0)),
        out_shape=jax.ShapeDtypeStruct((B, N, D), F32),
    )(rg.reshape(B, N, K, D), s, b20)

    return jnp.transpose(out, (0, 2, 1))


# speedup vs baseline: 11.0134x; 1.0142x over previous
import functools

import jax
import jax.numpy as jnp
from jax import lax
from jax.experimental import pallas as pl
from jax.experimental.pallas import tpu as pltpu
from jax.experimental.pallas import tpu_sc as plsc

B, N, K, D = 8, 2048, 8, 128
RT = 128
NT = N // RT
F32 = jnp.float32
_HI = lax.Precision.HIGHEST


def _knn_proj_kernel(pc_all_ref, pc_rows_ref, xyzt_rows_ref, f_rows_ref,
                     wx1_ref, wf1_ref, idx_ref, g1_ref, p_ref):
    b = pl.program_id(0)
    xyz_all = pc_all_ref[0]
    a = pc_rows_ref[0]
    at = xyzt_rows_ref[0]
    f = f_rows_ref[0]
    sq_all = jnp.sum(xyz_all * xyz_all, axis=0, keepdims=True)
    sq_r = jnp.sum(at * at, axis=1, keepdims=True)
    inner = lax.dot_general(a.astype(jnp.bfloat16),
                            xyz_all.astype(jnp.bfloat16),
                            (((0,), (0,)), ((), ())),
                            preferred_element_type=F32)
    dist = (sq_r - 2.0 * inner) + sq_all
    col = lax.broadcasted_iota(jnp.int32, (RT, N), 1)
    base = b * N
    picks = []
    for _ in range(K):
        mn = jnp.min(dist, axis=1, keepdims=True)
        am = jnp.min(jnp.where(dist == mn, col, N), axis=1, keepdims=True)
        picks.append(am + base)
        dist = jnp.where(col == am, jnp.inf, dist)
    idx_ref[0] = jnp.concatenate(picks, axis=1)
    p = lax.dot_general(a, wx1_ref[...], (((0,), (0,)), ((), ())),
                        preferred_element_type=F32, precision=_HI)
    g = p + lax.dot_general(f.astype(jnp.bfloat16),
                            wf1_ref[...].astype(jnp.bfloat16),
                            (((0,), (0,)), ((), ())),
                            preferred_element_type=F32)
    p_ref[0] = p
    g1_ref[0] = g


def _sa1_kernel(hg_ref, p_ref, pc_ref, w1_ref, w2_ref, b0_ref, b1_ref,
                b2_ref, wx2_ref, wf2_ref, r_ref, s_ref):
    hg = hg_ref[0]
    p = p_ref[0]
    h0 = hg - p[:, None, :] + b0_ref[...].reshape(1, 1, D)
    mu0 = jnp.mean(h0, axis=(0, 1), keepdims=True)
    inv0 = lax.rsqrt(jnp.mean((h0 - mu0) ** 2, axis=(0, 1), keepdims=True)
                     + 1e-5)
    a0 = jnp.maximum((h0 - mu0) * inv0, 0.0).reshape(N * K, D)
    h1 = jnp.dot(a0, w1_ref[...], preferred_element_type=F32,
                 precision=_HI) + b1_ref[...]
    mu1 = jnp.mean(h1, axis=0, keepdims=True)
    inv1 = lax.rsqrt(jnp.mean((h1 - mu1) ** 2, axis=0, keepdims=True) + 1e-5)
    a1 = jnp.maximum((h1 - mu1) * inv1, 0.0)
    h2 = jnp.dot(a1, w2_ref[...], preferred_element_type=F32,
                 precision=_HI) + b2_ref[...]
    mu2 = jnp.mean(h2, axis=0, keepdims=True)
    inv2 = lax.rsqrt(jnp.mean((h2 - mu2) ** 2, axis=0, keepdims=True) + 1e-5)
    m2 = jnp.max(h2.reshape(N, K, D), axis=1)
    fl1 = jnp.maximum((m2 - mu2) * inv2, 0.0)
    xyz = pc_ref[0]
    s = lax.dot_general(xyz, wx2_ref[...], (((0,), (0,)), ((), ())),
                        preferred_element_type=F32, precision=_HI)
    r = s + jnp.dot(fl1.astype(jnp.bfloat16),
                    wf2_ref[...].astype(jnp.bfloat16),
                    preferred_element_type=F32)
    r_ref[0] = r
    s_ref[0] = s


def _sa2_kernel(rg_ref, s_ref, b20_ref, out_ref):
    rg = rg_ref[0]
    s = s_ref[0]
    h = rg - s[:, None, :] + b20_ref[...].reshape(1, 1, D)
    mu = jnp.mean(h, axis=(0, 1), keepdims=True)
    inv = lax.rsqrt(jnp.mean((h - mu) ** 2, axis=(0, 1), keepdims=True) + 1e-5)
    m = jnp.max(h, axis=1)
    out_ref[0] = jnp.tanh((m - mu[0]) * inv[0])


def _sc_gather(table, idx2d):
    info = plsc.get_sparse_core_info()
    nw = info.num_cores * info.num_subcores
    chunks = idx2d.shape[0]
    cpw = chunks // nw
    mesh = plsc.VectorSubcoreMesh(core_axis_name="c", subcore_axis_name="s")

    @functools.partial(
        pl.kernel, mesh=mesh,
        out_type=jax.ShapeDtypeStruct((chunks * 128, D), F32),
        scratch_types=[
            pltpu.VMEM((cpw, 128), jnp.int32),
            pltpu.VMEM((128, D), F32),
            pltpu.SemaphoreType.DMA,
        ],
    )
    def k(table_hbm, idx_hbm, out_hbm, idx_v, rows_v, sem):
        wid = lax.axis_index("s") * info.num_cores + lax.axis_index("c")
        pltpu.sync_copy(idx_hbm.at[pl.ds(wid * cpw, cpw)], idx_v)

        def body(c, _):
            pltpu.async_copy(table_hbm.at[idx_v.at[c]], rows_v, sem).wait()
            pltpu.sync_copy(rows_v,
                            out_hbm.at[pl.ds((wid * cpw + c) * 128, 128)])
            return 0

        lax.fori_loop(0, cpw, body, 0)

    return k(table, idx2d)


def kernel(pc, feature, sa1_w0, sa1_b0, sa1_w1, sa1_b1, sa1_w2, sa1_b2,
           sa2_w0, sa2_b0):
    wx1, wf1 = sa1_w0[:3], sa1_w0[3:]
    wx2, wf2 = sa2_w0[:3], sa2_w0[3:]
    b0 = sa1_b0.reshape(1, D)
    b1 = sa1_b1.reshape(1, D)
    b2 = sa1_b2.reshape(1, D)
    b20 = sa2_b0.reshape(1, D)

    xyzt = jnp.transpose(pc, (0, 2, 1))
    idx, g1, p = pl.pallas_call(
        _knn_proj_kernel,
        grid=(B, NT),
        in_specs=[
            pl.BlockSpec((1, 3, N), lambda b, t: (b, 0, 0)),
            pl.BlockSpec((1, 3, RT), lambda b, t: (b, 0, t)),
            pl.BlockSpec((1, RT, 3), lambda b, t: (b, t, 0)),
            pl.BlockSpec((1, 64, RT), lambda b, t: (b, 0, t)),
            pl.BlockSpec((3, D), lambda b, t: (0, 0)),
            pl.BlockSpec((64, D), lambda b, t: (0, 0)),
        ],
        out_specs=[
            pl.BlockSpec((1, RT, K), lambda b, t: (b, t, 0)),
            pl.BlockSpec((1, RT, D), lambda b, t: (b, t, 0)),
            pl.BlockSpec((1, RT, D), lambda b, t: (b, t, 0)),
        ],
        out_shape=[
            jax.ShapeDtypeStruct((B, N, K), jnp.int32),
            jax.ShapeDtypeStruct((B, N, D), F32),
            jax.ShapeDtypeStruct((B, N, D), F32),
        ],
    )(pc, pc, xyzt, feature, wx1, wf1)

    idx2d = idx.reshape(B * N * K // 128, 128)
    hg = _sc_gather(g1.reshape(B * N, D), idx2d)

    full = lambda shape: pl.BlockSpec(shape, lambda b: (0,) * len(shape))
    r, s = pl.pallas_call(
        _sa1_kernel,
        grid=(B,),
        in_specs=[
            pl.BlockSpec((1, N, K, D), lambda b: (b, 0, 0, 0)),
            pl.BlockSpec((1, N, D), lambda b: (b, 0, 0)),
            pl.BlockSpec((1, 3, N), lambda b: (b, 0, 0)),
            full((D, D)), full((D, D)), full((1, D)), full((1, D)),
            full((1, D)), full((3, D)), full((D, D)),
        ],
        out_specs=[
            pl.BlockSpec((1, N, D), lambda b: (b, 0, 0)),
            pl.BlockSpec((1, N, D), lambda b: (b, 0, 0)),
        ],
        out_shape=[
            jax.ShapeDtypeStruct((B, N, D), F32),
            jax.ShapeDtypeStruct((B, N, D), F32),
        ],
    )(hg.reshape(B, N, K, D), p, pc, sa1_w1, sa1_w2, b0, b1, b2, wx2, wf2)

    rg = _sc_gather(r.reshape(B * N, D), idx2d)

    out = pl.pallas_call(
        _sa2_kernel,
        grid=(B,),
        in_specs=[
            pl.BlockSpec((1, N, K, D), lambda b: (b, 0, 0, 0)),
            pl.BlockSpec((1, N, D), lambda b: (b, 0, 0)),
            full((1, D)),
        ],
        out_specs=pl.BlockSpec((1, N, D), lambda b: (b, 0, 0)),
        out_shape=jax.ShapeDtypeStruct((B, N, D), F32),
    )(rg.reshape(B, N, K, D), s, b20)

    return jnp.transpose(out, (0, 2, 1))


# speedup vs baseline: 17.9686x; 1.6315x over previous
import functools

import jax
import jax.numpy as jnp
from jax import lax
from jax.experimental import pallas as pl
from jax.experimental.pallas import tpu as pltpu
from jax.experimental.pallas import tpu_sc as plsc

B, N, K, D = 8, 2048, 8, 128
RT = 256
NT = N // RT
F32 = jnp.float32
_HI = lax.Precision.HIGHEST


def _knn_proj_kernel(pc_all_ref, pc_rows_ref, xyzt_rows_ref, f_rows_ref,
                     wx1_ref, wf1_ref, idx_ref, g1_ref, p_ref):
    b = pl.program_id(0)
    xyz_all = pc_all_ref[0]
    a = pc_rows_ref[0]
    at = xyzt_rows_ref[0]
    f = f_rows_ref[0]
    sq_all = jnp.sum(xyz_all * xyz_all, axis=0, keepdims=True)
    sq_r = jnp.sum(at * at, axis=1, keepdims=True)
    inner = lax.dot_general(a.astype(jnp.bfloat16),
                            xyz_all.astype(jnp.bfloat16),
                            (((0,), (0,)), ((), ())),
                            preferred_element_type=F32)
    dist = (sq_r - 2.0 * inner) + sq_all
    col = lax.broadcasted_iota(jnp.int32, (RT, N), 1)
    base = b * N
    picks = []
    for _ in range(K):
        am = jnp.argmin(dist, axis=1).astype(jnp.int32)[:, None]
        picks.append(am + base)
        dist = jnp.where(col == am, jnp.inf, dist)
    idx_ref[0] = jnp.concatenate(picks, axis=1)
    p = lax.dot_general(a, wx1_ref[...], (((0,), (0,)), ((), ())),
                        preferred_element_type=F32, precision=_HI)
    g = p + lax.dot_general(f.astype(jnp.bfloat16),
                            wf1_ref[...].astype(jnp.bfloat16),
                            (((0,), (0,)), ((), ())),
                            preferred_element_type=F32)
    p_ref[0] = p
    g1_ref[0] = g


def _sa1_kernel(hg_ref, p_ref, pc_ref, w1_ref, w2_ref, b0_ref, b1_ref,
                b2_ref, wx2_ref, wf2_ref, r_ref, s_ref):
    hg = hg_ref[0]
    p = p_ref[0]
    h0 = hg - p[:, None, :] + b0_ref[...].reshape(1, 1, D)
    mu0 = jnp.mean(h0, axis=(0, 1), keepdims=True)
    inv0 = lax.rsqrt(jnp.mean((h0 - mu0) ** 2, axis=(0, 1), keepdims=True)
                     + 1e-5)
    a0 = jnp.maximum((h0 - mu0) * inv0, 0.0).reshape(N * K, D)
    h1 = jnp.dot(a0.astype(jnp.bfloat16), w1_ref[...].astype(jnp.bfloat16),
                 preferred_element_type=F32) + b1_ref[...]
    mu1 = jnp.mean(h1, axis=0, keepdims=True)
    inv1 = lax.rsqrt(jnp.mean((h1 - mu1) ** 2, axis=0, keepdims=True) + 1e-5)
    a1 = jnp.maximum((h1 - mu1) * inv1, 0.0)
    h2 = jnp.dot(a1.astype(jnp.bfloat16), w2_ref[...].astype(jnp.bfloat16),
                 preferred_element_type=F32) + b2_ref[...]
    mu2 = jnp.mean(h2, axis=0, keepdims=True)
    inv2 = lax.rsqrt(jnp.mean((h2 - mu2) ** 2, axis=0, keepdims=True) + 1e-5)
    m2 = jnp.max(h2.reshape(N, K, D), axis=1)
    fl1 = jnp.maximum((m2 - mu2) * inv2, 0.0)
    xyz = pc_ref[0]
    s = lax.dot_general(xyz, wx2_ref[...], (((0,), (0,)), ((), ())),
                        preferred_element_type=F32, precision=_HI)
    r = s + jnp.dot(fl1.astype(jnp.bfloat16),
                    wf2_ref[...].astype(jnp.bfloat16),
                    preferred_element_type=F32)
    r_ref[0] = r
    s_ref[0] = s


def _sa2_kernel(rg_ref, s_ref, b20_ref, out_ref):
    rg = rg_ref[0]
    s = s_ref[0]
    h = rg - s[:, None, :] + b20_ref[...].reshape(1, 1, D)
    mu = jnp.mean(h, axis=(0, 1), keepdims=True)
    inv = lax.rsqrt(jnp.mean((h - mu) ** 2, axis=(0, 1), keepdims=True) + 1e-5)
    m = jnp.max(h, axis=1)
    out_ref[0] = jnp.tanh((m - mu[0]) * inv[0])


def _sc_gather(table, idx2d):
    info = plsc.get_sparse_core_info()
    nw = info.num_cores * info.num_subcores
    chunks = idx2d.shape[0]
    cpw = chunks // nw
    mesh = plsc.VectorSubcoreMesh(core_axis_name="c", subcore_axis_name="s")

    @functools.partial(
        pl.kernel, mesh=mesh,
        out_type=jax.ShapeDtypeStruct((chunks * 128, D), F32),
        scratch_types=[
            pltpu.VMEM((cpw, 128), jnp.int32),
            pltpu.VMEM((128, D), F32),
            pltpu.SemaphoreType.DMA,
        ],
    )
    def k(table_hbm, idx_hbm, out_hbm, idx_v, rows_v, sem):
        wid = lax.axis_index("s") * info.num_cores + lax.axis_index("c")
        pltpu.sync_copy(idx_hbm.at[pl.ds(wid * cpw, cpw)], idx_v)

        def body(c, _):
            pltpu.async_copy(table_hbm.at[idx_v.at[c]], rows_v, sem).wait()
            pltpu.sync_copy(rows_v,
                            out_hbm.at[pl.ds((wid * cpw + c) * 128, 128)])
            return 0

        lax.fori_loop(0, cpw, body, 0)

    return k(table, idx2d)


def kernel(pc, feature, sa1_w0, sa1_b0, sa1_w1, sa1_b1, sa1_w2, sa1_b2,
           sa2_w0, sa2_b0):
    wx1, wf1 = sa1_w0[:3], sa1_w0[3:]
    wx2, wf2 = sa2_w0[:3], sa2_w0[3:]
    b0 = sa1_b0.reshape(1, D)
    b1 = sa1_b1.reshape(1, D)
    b2 = sa1_b2.reshape(1, D)
    b20 = sa2_b0.reshape(1, D)

    xyzt = jnp.transpose(pc, (0, 2, 1))
    idx, g1, p = pl.pallas_call(
        _knn_proj_kernel,
        grid=(B, NT),
        in_specs=[
            pl.BlockSpec((1, 3, N), lambda b, t: (b, 0, 0)),
            pl.BlockSpec((1, 3, RT), lambda b, t: (b, 0, t)),
            pl.BlockSpec((1, RT, 3), lambda b, t: (b, t, 0)),
            pl.BlockSpec((1, 64, RT), lambda b, t: (b, 0, t)),
            pl.BlockSpec((3, D), lambda b, t: (0, 0)),
            pl.BlockSpec((64, D), lambda b, t: (0, 0)),
        ],
        out_specs=[
            pl.BlockSpec((1, RT, K), lambda b, t: (b, t, 0)),
            pl.BlockSpec((1, RT, D), lambda b, t: (b, t, 0)),
            pl.BlockSpec((1, RT, D), lambda b, t: (b, t, 0)),
        ],
        out_shape=[
            jax.ShapeDtypeStruct((B, N, K), jnp.int32),
            jax.ShapeDtypeStruct((B, N, D), F32),
            jax.ShapeDtypeStruct((B, N, D), F32),
        ],
    )(pc, pc, xyzt, feature, wx1, wf1)

    idx2d = idx.reshape(B * N * K // 128, 128)
    hg = _sc_gather(g1.reshape(B * N, D), idx2d)

    full = lambda shape: pl.BlockSpec(shape, lambda b: (0,) * len(shape))
    r, s = pl.pallas_call(
        _sa1_kernel,
        grid=(B,),
        in_specs=[
            pl.BlockSpec((1, N, K, D), lambda b: (b, 0, 0, 0)),
            pl.BlockSpec((1, N, D), lambda b: (b, 0, 0)),
            pl.BlockSpec((1, 3, N), lambda b: (b, 0, 0)),
            full((D, D)), full((D, D)), full((1, D)), full((1, D)),
            full((1, D)), full((3, D)), full((D, D)),
        ],
        out_specs=[
            pl.BlockSpec((1, N, D), lambda b: (b, 0, 0)),
            pl.BlockSpec((1, N, D), lambda b: (b, 0, 0)),
        ],
        out_shape=[
            jax.ShapeDtypeStruct((B, N, D), F32),
            jax.ShapeDtypeStruct((B, N, D), F32),
        ],
    )(hg.reshape(B, N, K, D), p, pc, sa1_w1, sa1_w2, b0, b1, b2, wx2, wf2)

    rg = _sc_gather(r.reshape(B * N, D), idx2d)

    out = pl.pallas_call(
        _sa2_kernel,
        grid=(B,),
        in_specs=[
            pl.BlockSpec((1, N, K, D), lambda b: (b, 0, 0, 0)),
            pl.BlockSpec((1, N, D), lambda b: (b, 0, 0)),
            full((1, D)),
        ],
        out_specs=pl.BlockSpec((1, N, D), lambda b: (b, 0, 0)),
        out_shape=jax.ShapeDtypeStruct((B, N, D), F32),
    )(rg.reshape(B, N, K, D), s, b20)

    return jnp.transpose(out, (0, 2, 1))


# speedup vs baseline: 19.1480x; 1.0656x over previous
import functools

import jax
import jax.numpy as jnp
from jax import lax
from jax.experimental import pallas as pl
from jax.experimental.pallas import tpu as pltpu
from jax.experimental.pallas import tpu_sc as plsc

B, N, K, D = 8, 2048, 8, 128
RT = 256
NT = N // RT
F32 = jnp.float32
_HI = lax.Precision.HIGHEST


def _knn_proj_kernel(pc_all_ref, pc_rows_ref, xyzt_rows_ref, f_rows_ref,
                     wx1_ref, wf1_ref, idx_ref, g1_ref, p_ref):
    b = pl.program_id(0)
    xyz_all = pc_all_ref[0]
    a = pc_rows_ref[0]
    at = xyzt_rows_ref[0]
    f = f_rows_ref[0]
    sq_all = jnp.sum(xyz_all * xyz_all, axis=0, keepdims=True)
    sq_r = jnp.sum(at * at, axis=1, keepdims=True)
    inner = lax.dot_general(a.astype(jnp.bfloat16),
                            xyz_all.astype(jnp.bfloat16),
                            (((0,), (0,)), ((), ())),
                            preferred_element_type=F32)
    dist = (sq_r - 2.0 * inner) + sq_all
    col = lax.broadcasted_iota(jnp.int32, (RT, N), 1)
    base = b * N
    picks = []
    for _ in range(K):
        am = jnp.argmin(dist, axis=1).astype(jnp.int32)[:, None]
        picks.append(am + base)
        dist = jnp.where(col == am, jnp.inf, dist)
    idx_ref[0] = jnp.concatenate(picks, axis=1)
    p = lax.dot_general(a, wx1_ref[...], (((0,), (0,)), ((), ())),
                        preferred_element_type=F32, precision=_HI)
    g = p + lax.dot_general(f.astype(jnp.bfloat16),
                            wf1_ref[...].astype(jnp.bfloat16),
                            (((0,), (0,)), ((), ())),
                            preferred_element_type=F32)
    p_ref[0] = p
    g1_ref[0] = g


def _sa1_kernel(hg_ref, p_ref, pc_ref, w1_ref, w2_ref, b0_ref, b1_ref,
                b2_ref, wx2_ref, wf2_ref, r_ref, s_ref):
    hg = hg_ref[0]
    p = p_ref[0]
    h0 = hg - p[:, None, :] + b0_ref[...].reshape(1, 1, D)
    mu0 = jnp.mean(h0, axis=(0, 1), keepdims=True)
    inv0 = lax.rsqrt(jnp.mean((h0 - mu0) ** 2, axis=(0, 1), keepdims=True)
                     + 1e-5)
    a0 = jnp.maximum((h0 - mu0) * inv0, 0.0).reshape(N * K, D)
    h1 = jnp.dot(a0.astype(jnp.bfloat16), w1_ref[...].astype(jnp.bfloat16),
                 preferred_element_type=F32) + b1_ref[...]
    mu1 = jnp.mean(h1, axis=0, keepdims=True)
    inv1 = lax.rsqrt(jnp.mean((h1 - mu1) ** 2, axis=0, keepdims=True) + 1e-5)
    a1 = jnp.maximum((h1 - mu1) * inv1, 0.0)
    h2 = jnp.dot(a1.astype(jnp.bfloat16), w2_ref[...].astype(jnp.bfloat16),
                 preferred_element_type=F32) + b2_ref[...]
    mu2 = jnp.mean(h2, axis=0, keepdims=True)
    inv2 = lax.rsqrt(jnp.mean((h2 - mu2) ** 2, axis=0, keepdims=True) + 1e-5)
    m2 = jnp.max(h2.reshape(N, K, D), axis=1)
    fl1 = jnp.maximum((m2 - mu2) * inv2, 0.0)
    xyz = pc_ref[0]
    s = lax.dot_general(xyz, wx2_ref[...], (((0,), (0,)), ((), ())),
                        preferred_element_type=F32, precision=_HI)
    r = s + jnp.dot(fl1.astype(jnp.bfloat16),
                    wf2_ref[...].astype(jnp.bfloat16),
                    preferred_element_type=F32)
    r_ref[0] = r
    s_ref[0] = s


def _sa2_kernel(rg_ref, s_ref, b20_ref, out_ref):
    rg = rg_ref[0]
    s = s_ref[0]
    h = rg - s[:, None, :] + b20_ref[...].reshape(1, 1, D)
    mu = jnp.mean(h, axis=(0, 1), keepdims=True)
    inv = lax.rsqrt(jnp.mean((h - mu) ** 2, axis=(0, 1), keepdims=True) + 1e-5)
    m = jnp.max(h, axis=1)
    out_ref[0] = jnp.tanh((m - mu[0]) * inv[0])


def _sc_gather(table, idx2d):
    info = plsc.get_sparse_core_info()
    nw = info.num_cores * info.num_subcores
    chunks = idx2d.shape[0]
    cpw = chunks // nw
    mesh = plsc.VectorSubcoreMesh(core_axis_name="c", subcore_axis_name="s")

    @functools.partial(
        pl.kernel, mesh=mesh,
        out_type=jax.ShapeDtypeStruct((chunks * 128, D), F32),
        scratch_types=[
            pltpu.VMEM((cpw + 2, 128), jnp.int32),
            pltpu.VMEM((128, D), F32),
            pltpu.VMEM((128, D), F32),
            pltpu.SemaphoreType.DMA,
            pltpu.SemaphoreType.DMA,
        ],
    )
    def k(table_hbm, idx_hbm, out_hbm, idx_v, rows0, rows1, sem0, sem1):
        wid = lax.axis_index("s") * info.num_cores + lax.axis_index("c")
        pltpu.sync_copy(idx_hbm.at[pl.ds(wid * cpw, cpw)],
                        idx_v.at[pl.ds(0, cpw)])
        pltpu.sync_copy(idx_hbm.at[pl.ds(wid * cpw, 2)],
                        idx_v.at[pl.ds(cpw, 2)])
        pltpu.async_copy(table_hbm.at[idx_v.at[0]], rows0, sem0)
        pltpu.async_copy(table_hbm.at[idx_v.at[1]], rows1, sem1)

        def body(i, _):
            c0 = 2 * i
            pltpu.make_async_copy(table_hbm.at[idx_v.at[c0]], rows0,
                                  sem0).wait()
            pltpu.sync_copy(rows0,
                            out_hbm.at[pl.ds((wid * cpw + c0) * 128, 128)])
            pltpu.async_copy(table_hbm.at[idx_v.at[c0 + 2]], rows0, sem0)
            pltpu.make_async_copy(table_hbm.at[idx_v.at[c0 + 1]], rows1,
                                  sem1).wait()
            pltpu.sync_copy(rows1,
                            out_hbm.at[pl.ds((wid * cpw + c0 + 1) * 128, 128)])
            pltpu.async_copy(table_hbm.at[idx_v.at[c0 + 3]], rows1, sem1)
            return 0

        lax.fori_loop(0, cpw // 2, body, 0)
        pltpu.make_async_copy(table_hbm.at[idx_v.at[cpw]], rows0, sem0).wait()
        pltpu.make_async_copy(table_hbm.at[idx_v.at[cpw + 1]], rows1,
                              sem1).wait()

    return k(table, idx2d)


def kernel(pc, feature, sa1_w0, sa1_b0, sa1_w1, sa1_b1, sa1_w2, sa1_b2,
           sa2_w0, sa2_b0):
    wx1, wf1 = sa1_w0[:3], sa1_w0[3:]
    wx2, wf2 = sa2_w0[:3], sa2_w0[3:]
    b0 = sa1_b0.reshape(1, D)
    b1 = sa1_b1.reshape(1, D)
    b2 = sa1_b2.reshape(1, D)
    b20 = sa2_b0.reshape(1, D)

    xyzt = jnp.transpose(pc, (0, 2, 1))
    idx, g1, p = pl.pallas_call(
        _knn_proj_kernel,
        grid=(B, NT),
        in_specs=[
            pl.BlockSpec((1, 3, N), lambda b, t: (b, 0, 0)),
            pl.BlockSpec((1, 3, RT), lambda b, t: (b, 0, t)),
            pl.BlockSpec((1, RT, 3), lambda b, t: (b, t, 0)),
            pl.BlockSpec((1, 64, RT), lambda b, t: (b, 0, t)),
            pl.BlockSpec((3, D), lambda b, t: (0, 0)),
            pl.BlockSpec((64, D), lambda b, t: (0, 0)),
        ],
        out_specs=[
            pl.BlockSpec((1, RT, K), lambda b, t: (b, t, 0)),
            pl.BlockSpec((1, RT, D), lambda b, t: (b, t, 0)),
            pl.BlockSpec((1, RT, D), lambda b, t: (b, t, 0)),
        ],
        out_shape=[
            jax.ShapeDtypeStruct((B, N, K), jnp.int32),
            jax.ShapeDtypeStruct((B, N, D), F32),
            jax.ShapeDtypeStruct((B, N, D), F32),
        ],
    )(pc, pc, xyzt, feature, wx1, wf1)

    idx2d = idx.reshape(B * N * K // 128, 128)
    hg = _sc_gather(g1.reshape(B * N, D), idx2d)

    full = lambda shape: pl.BlockSpec(shape, lambda b: (0,) * len(shape))
    r, s = pl.pallas_call(
        _sa1_kernel,
        grid=(B,),
        in_specs=[
            pl.BlockSpec((1, N, K, D), lambda b: (b, 0, 0, 0)),
            pl.BlockSpec((1, N, D), lambda b: (b, 0, 0)),
            pl.BlockSpec((1, 3, N), lambda b: (b, 0, 0)),
            full((D, D)), full((D, D)), full((1, D)), full((1, D)),
            full((1, D)), full((3, D)), full((D, D)),
        ],
        out_specs=[
            pl.BlockSpec((1, N, D), lambda b: (b, 0, 0)),
            pl.BlockSpec((1, N, D), lambda b: (b, 0, 0)),
        ],
        out_shape=[
            jax.ShapeDtypeStruct((B, N, D), F32),
            jax.ShapeDtypeStruct((B, N, D), F32),
        ],
    )(hg.reshape(B, N, K, D), p, pc, sa1_w1, sa1_w2, b0, b1, b2, wx2, wf2)

    rg = _sc_gather(r.reshape(B * N, D), idx2d)

    out = pl.pallas_call(
        _sa2_kernel,
        grid=(B,),
        in_specs=[
            pl.BlockSpec((1, N, K, D), lambda b: (b, 0, 0, 0)),
            pl.BlockSpec((1, N, D), lambda b: (b, 0, 0)),
            full((1, D)),
        ],
        out_specs=pl.BlockSpec((1, N, D), lambda b: (b, 0, 0)),
        out_shape=jax.ShapeDtypeStruct((B, N, D), F32),
    )(rg.reshape(B, N, K, D), s, b20)

    return jnp.transpose(out, (0, 2, 1))
